# baseline (speedup 1.0000x reference)
import jax
import jax.numpy as jnp
from jax.experimental import pallas as pl


def _copy_body(x_ref, o_ref):
    o_ref[...] = x_ref[...]


def _pl_copy(x):
    return pl.pallas_call(
        _copy_body,
        out_shape=jax.ShapeDtypeStruct(x.shape, x.dtype),
    )(x)


def _gcn_layer(h, src, dst, ew, W, b, n):
    deg = jax.ops.segment_sum(ew, dst, num_segments=n) + 1e-9
    norm = ew / jnp.sqrt(deg[src] * deg[dst])
    msg = h[src] * norm[:, None]
    agg = jax.ops.segment_sum(msg, dst, num_segments=n)
    return agg @ W + b


def _encoder(x, edge_index, ew, W1, b1, W2, b2):
    src = edge_index[0]
    dst = edge_index[1]
    n = x.shape[0]
    h = jax.nn.relu(_gcn_layer(x, src, dst, ew, W1, b1, n))
    h = _gcn_layer(h, src, dst, ew, W2, b2, n)
    return h


def kernel(x, edge_index, edge_weight, W1, b1, W2, b2):
    k = jax.random.key(42)
    k1, k2, k3, k4 = jax.random.split(k, 4)
    keep_e1 = jax.random.bernoulli(k1, 0.7, (edge_weight.shape[0],)).astype(jnp.float32)
    keep_e2 = jax.random.bernoulli(k2, 0.7, (edge_weight.shape[0],)).astype(jnp.float32)
    fm1 = jax.random.bernoulli(k3, 0.7, (x.shape[1],)).astype(jnp.float32)
    fm2 = jax.random.bernoulli(k4, 0.7, (x.shape[1],)).astype(jnp.float32)
    z = _encoder(x, edge_index, edge_weight, W1, b1, W2, b2)
    z1 = _encoder(x * fm1[None, :], edge_index, edge_weight * keep_e1, W1, b1, W2, b2)
    z2 = _encoder(x * fm2[None, :], edge_index, edge_weight * keep_e2, W1, b1, W2, b2)
    return (_pl_copy(z), _pl_copy(z1), _pl_copy(z2))


# speedup vs baseline: 4.6013x; 4.6013x over previous
import functools

import jax
import jax.numpy as jnp
from jax import lax
from jax.experimental import pallas as pl
from jax.experimental.pallas import tpu as pltpu
from jax.experimental.pallas import tpu_sc as plsc

N = 10000
E = 320000
D = 128

NC = 2
NS = 16
L = 16
NW = NC * NS

K = 128
CH = (E + NW * K - 1) // (NW * K) * NW
CPT = CH // NW
EPAD = CH * K

NPAD = 10240
RPT = NPAD // NS

_mesh = plsc.VectorSubcoreMesh(
    core_axis_name="c", subcore_axis_name="s", num_cores=NC, num_subcores=NS)

_i32 = jnp.int32
_f32 = jnp.float32


def _deg_body(dst1, ew1, k11, k21, deg_out,
              h0, h1, h2, dst_v, ew_v, k1_v, k2_v, red_v, acc_v, stage_sh):
    c_ax = lax.axis_index("c")
    s_ax = lax.axis_index("s")
    wid = c_ax * NS + s_ax

    z16 = jnp.zeros((L,), _f32)

    def zero_body(i, _):
        h0[pl.ds(i * L, L)] = z16
        h1[pl.ds(i * L, L)] = z16
        h2[pl.ds(i * L, L)] = z16
        return 0
    lax.fori_loop(0, NPAD // L, zero_body, 0)

    def chunk_body(i, _):
        base = (wid * CPT + i) * K
        pltpu.sync_copy(dst1.at[pl.ds(base, K)], dst_v)
        pltpu.sync_copy(ew1.at[pl.ds(base, K)], ew_v)
        pltpu.sync_copy(k11.at[pl.ds(base, K)], k1_v)
        pltpu.sync_copy(k21.at[pl.ds(base, K)], k2_v)

        def grp(g, _):
            sl = pl.ds(g * L, L)
            d16 = dst_v[sl]
            w16 = ew_v[sl]
            plsc.addupdate_scatter(h0, [d16], w16)
            plsc.addupdate_scatter(h1, [d16], w16 * k1_v[sl])
            plsc.addupdate_scatter(h2, [d16], w16 * k2_v[sl])
            return 0
        lax.fori_loop(0, K // L, grp, 0)
        return 0
    lax.fori_loop(0, CPT, chunk_body, 0)

    pltpu.sync_copy(h0, stage_sh.at[pl.ds((s_ax * 3 + 0) * NPAD, NPAD)])
    pltpu.sync_copy(h1, stage_sh.at[pl.ds((s_ax * 3 + 1) * NPAD, NPAD)])
    pltpu.sync_copy(h2, stage_sh.at[pl.ds((s_ax * 3 + 2) * NPAD, NPAD)])
    plsc.subcore_barrier()

    for v in range(3):
        pltpu.sync_copy(stage_sh.at[pl.ds(v * NPAD + s_ax * RPT, RPT)], acc_v)

        def red_t(t, _):
            pltpu.sync_copy(
                stage_sh.at[pl.ds((t * 3 + v) * NPAD + s_ax * RPT, RPT)], red_v)

            def addg(g, _):
                sl = pl.ds(g * L, L)
                acc_v[sl] = acc_v[sl] + red_v[sl]
                return 0
            lax.fori_loop(0, RPT // L, addg, 0)
            return 0
        lax.fori_loop(1, NS, red_t, 0)
        pltpu.sync_copy(
            acc_v,
            deg_out.at[pl.ds((c_ax * 3 + v) * NPAD + s_ax * RPT, RPT)])


_deg_kernel = functools.partial(
    pl.kernel,
    out_type=jax.ShapeDtypeStruct((NC * 3 * NPAD,), _f32),
    mesh=_mesh,
    compiler_params=pltpu.CompilerParams(needs_layout_passes=False),
    scratch_types=[
        pltpu.VMEM((NPAD,), _f32),
        pltpu.VMEM((NPAD,), _f32),
        pltpu.VMEM((NPAD,), _f32),
        pltpu.VMEM((K,), _i32),
        pltpu.VMEM((K,), _f32),
        pltpu.VMEM((K,), _f32),
        pltpu.VMEM((K,), _f32),
        pltpu.VMEM((RPT,), _f32),
        pltpu.VMEM((RPT,), _f32),
        pltpu.VMEM_SHARED((NS * 3 * NPAD,), _f32),
    ],
)(_deg_body)


def _prop_body(table, src1, dst1, ew1, keep1d, dinv_hbm, part_out,
               dinv_v, src_v, dst_v, ew_v, keep_v, c_v, rows_v, acc_sh, sem):
    c_ax = lax.axis_index("c")
    s_ax = lax.axis_index("s")
    wid = c_ax * NS + s_ax

    pltpu.sync_copy(dinv_hbm, dinv_v)

    z16 = jnp.zeros((L,), _f32)
    zi16 = jnp.zeros((L,), _i32)

    def zrow(k, _):
        for jj in range(D // L):
            rows_v[k, pl.ds(jj * L, L)] = z16
        return 0
    lax.fori_loop(0, K, zrow, 0)
    for q in range(RPT // K):
        pltpu.sync_copy(rows_v, acc_sh.at[pl.ds(s_ax * RPT + q * K, K)])
    plsc.subcore_barrier()

    def chunk_body(i, _):
        base = (wid * CPT + i) * K
        pltpu.sync_copy(src1.at[pl.ds(base, K)], src_v)
        pltpu.sync_copy(dst1.at[pl.ds(base, K)], dst_v)
        pltpu.sync_copy(ew1.at[pl.ds(base, K)], ew_v)
        pltpu.sync_copy(keep1d.at[pl.ds(base, K)], keep_v)
        pltpu.async_copy(table.at[src_v], rows_v, sem).wait()

        def cgrp(g, _):
            sl = pl.ds(g * L, L)
            dv = plsc.load_gather(dinv_v, [src_v[sl]])
            c_v[sl] = ew_v[sl] * keep_v[sl] * dv
            return 0
        lax.fori_loop(0, K // L, cgrp, 0)

        def scale(k, _):
            cb = plsc.load_gather(c_v, [zi16 + k])
            for jj in range(D // L):
                sl = pl.ds(jj * L, L)
                rows_v[k, sl] = rows_v[k, sl] * cb
            return 0
        lax.fori_loop(0, K, scale, 0)

        pltpu.sync_copy(rows_v, acc_sh.at[dst_v], add=True)
        return 0
    lax.fori_loop(0, CPT, chunk_body, 0)
    plsc.subcore_barrier()

    for q in range(RPT // K):
        base = s_ax * RPT + q * K
        pltpu.sync_copy(acc_sh.at[pl.ds(base, K)], rows_v)

        def wscale(k, _):
            db = plsc.load_gather(dinv_v, [zi16 + base + k])
            for jj in range(D // L):
                sl = pl.ds(jj * L, L)
                rows_v[k, sl] = rows_v[k, sl] * db
            return 0
        lax.fori_loop(0, K, wscale, 0)
        pltpu.sync_copy(rows_v, part_out.at[pl.ds(c_ax * NPAD + base, K)])


_prop_kernel = functools.partial(
    pl.kernel,
    out_type=jax.ShapeDtypeStruct((NC * NPAD, D), _f32),
    mesh=_mesh,
    compiler_params=pltpu.CompilerParams(needs_layout_passes=False),
    scratch_types=[
        pltpu.VMEM((NPAD,), _f32),
        pltpu.VMEM((K,), _i32),
        pltpu.VMEM((K,), _i32),
        pltpu.VMEM((K,), _f32),
        pltpu.VMEM((K,), _f32),
        pltpu.VMEM((K,), _f32),
        pltpu.VMEM((K, D), _f32),
        pltpu.VMEM_SHARED((NPAD, D), _f32),
        pltpu.SemaphoreType.DMA,
    ],
)(_prop_body)


def _prep_body(deg_ref, dinv_ref):
    a = deg_ref[...]
    dinv_ref[...] = lax.rsqrt(a[0:3] + a[3:6] + 1e-9)


def _tc_prep(degparts):
    return pl.pallas_call(
        _prep_body,
        out_shape=jax.ShapeDtypeStruct((3, NPAD), _f32),
    )(degparts)


_BR = 512


def _dense_body(p0_ref, p1_ref, fm_ref, w_ref, b_ref, o_ref, *, relu):
    a = p0_ref[...] + p1_ref[...]
    a = a * fm_ref[...]
    o = jnp.dot(a, w_ref[...], preferred_element_type=_f32) + b_ref[...]
    if relu:
        o = jnp.maximum(o, 0.0)
    o_ref[...] = o


def _tc_dense(parts, fm, W, b, relu):
    body = functools.partial(_dense_body, relu=relu)
    nb = NPAD // _BR
    return pl.pallas_call(
        body,
        grid=(nb,),
        in_specs=[
            pl.BlockSpec((_BR, D), lambda i: (i, 0)),
            pl.BlockSpec((_BR, D), lambda i, nb=nb: (nb + i, 0)),
            pl.BlockSpec((1, D), lambda i: (0, 0)),
            pl.BlockSpec((D, D), lambda i: (0, 0)),
            pl.BlockSpec((1, D), lambda i: (0, 0)),
        ],
        out_specs=pl.BlockSpec((_BR, D), lambda i: (i, 0)),
        out_shape=jax.ShapeDtypeStruct((NPAD, D), _f32),
    )(parts, parts, fm, W, b)


def _pad1d(a, fill):
    pad = EPAD - E
    return jnp.concatenate([a, jnp.full((pad,), fill, a.dtype)])


def kernel(x, edge_index, edge_weight, W1, b1, W2, b2):
    k = jax.random.key(42)
    k1, k2, k3, k4 = jax.random.split(k, 4)
    keep1 = jax.random.bernoulli(k1, 0.7, (E,)).astype(_f32)
    keep2 = jax.random.bernoulli(k2, 0.7, (E,)).astype(_f32)
    fm1 = jax.random.bernoulli(k3, 0.7, (D,)).astype(_f32)
    fm2 = jax.random.bernoulli(k4, 0.7, (D,)).astype(_f32)

    src1 = _pad1d(edge_index[0], 0)
    dst1 = _pad1d(edge_index[1], N)
    ew1 = _pad1d(edge_weight, 0.0)
    ones1 = _pad1d(jnp.ones((E,), _f32), 0.0)
    k11 = _pad1d(keep1, 0.0)
    k21 = _pad1d(keep2, 0.0)

    xpad = jnp.concatenate([x, jnp.zeros((NPAD - N, D), _f32)])

    degparts = _deg_kernel(dst1, ew1, k11, k21).reshape(NC * 3, NPAD)
    dinv = _tc_prep(degparts)

    fm0r = jnp.ones((1, D), _f32)
    fm1r = fm1.reshape(1, D)
    fm2r = fm2.reshape(1, D)
    b1r = b1.reshape(1, D)
    b2r = b2.reshape(1, D)

    outs = []
    for keep1d_v, fmr, vi in ((ones1, fm0r, 0), (k11, fm1r, 1), (k21, fm2r, 2)):
        dv = dinv[vi]
        p1 = _prop_kernel(xpad, src1, dst1, ew1, keep1d_v, dv)
        h = _tc_dense(p1, fmr, W1, b1r, relu=True)
        p2 = _prop_kernel(h, src1, dst1, ew1, keep1d_v, dv)
        z = _tc_dense(p2, fm0r, W2, b2r, relu=False)
        outs.append(z[:N])

    return tuple(outs)
